# Initial kernel scaffold; baseline (speedup 1.0000x reference)
import jax
import jax.numpy as jnp
from jax.experimental import pallas as pl


def kernel(x, edge_index, batch, params):
    raise NotImplementedError("write your pallas kernel here")


# speedup vs baseline: 4.6152x; 4.6152x over previous
import functools
import jax
import jax.numpy as jnp
from jax import lax
from jax.experimental import pallas as pl
from jax.experimental.pallas import tpu as pltpu
from jax.experimental.pallas import tpu_sc as plsc

_N = 10000
_E = 320000
_D = 128
_G = 64
_NC = 2
_NS = 16
_NW = _NC * _NS


_EPW = _E // _NW
_CHK = 80
_NCHK = _EPW // _CHK
_ZR = 80
_NZC = _N // _ZR


def _seg_sum_body(h_hbm, src_hbm, dst_hbm, out_hbm, src_v, dst_v, rows_v,
                  znc_v, acc_sh, gsem):
  c = lax.axis_index("c")
  s = lax.axis_index("s")
  w = c * _NS + s
  nck = jnp.where(s < _NZC - _NS * (_NZC // _NS), _NZC // _NS + 1, _NZC // _NS)

  def zbody(i, carry):
    for j in range(_D // 16):
      znc_v[i, pl.ds(j * 16, 16)] = jnp.zeros((16,), jnp.float32)
    return carry

  lax.fori_loop(0, _ZR, zbody, 0)

  def zcopy(k, carry):
    pltpu.sync_copy(znc_v, acc_sh.at[pl.ds((s + _NS * k) * _ZR, _ZR)])
    return carry

  lax.fori_loop(0, nck, zcopy, 0)
  plsc.subcore_barrier()

  def body(j, carry):
    base = w * _EPW + j * _CHK
    pltpu.sync_copy(src_hbm.at[pl.ds(base, _CHK)], src_v)
    pltpu.sync_copy(dst_hbm.at[pl.ds(base, _CHK)], dst_v)
    pltpu.async_copy(h_hbm.at[src_v], rows_v, gsem).wait()
    pltpu.sync_copy(rows_v, acc_sh.at[dst_v], add=True)
    return carry

  lax.fori_loop(0, _NCHK, body, 0)
  plsc.subcore_barrier()

  def dcopy(k, carry):
    r0 = (s + _NS * k) * _ZR
    pltpu.sync_copy(acc_sh.at[pl.ds(r0, _ZR)], znc_v)
    pltpu.sync_copy(znc_v, out_hbm.at[c, pl.ds(r0, _ZR)])
    return carry

  lax.fori_loop(0, nck, dcopy, 0)


def _seg_sum(h, src, dst):
  mesh = plsc.VectorSubcoreMesh(core_axis_name="c", subcore_axis_name="s")
  return pl.kernel(
      _seg_sum_body,
      out_type=jax.ShapeDtypeStruct((_NC, _N, _D), jnp.float32),
      mesh=mesh,
      scratch_types=[
          pltpu.VMEM((_CHK,), jnp.int32),
          pltpu.VMEM((_CHK,), jnp.int32),
          pltpu.VMEM((_CHK, _D), jnp.float32),
          pltpu.VMEM((_ZR, _D), jnp.float32),
          pltpu.VMEM_SHARED((_N, _D), jnp.float32),
          pltpu.SemaphoreType.DMA,
      ],
  )(h, src, dst)


_PW = 320
_PSTRIDE = 312


def _pool_body(h0_hbm, h1_hbm, h2_hbm, batch_hbm, out_hbm, bat_v, rows_v,
               acc_v):
  c = lax.axis_index("c")
  s = lax.axis_index("s")
  w = c * _NS + s
  base = jnp.where(w == _NW - 1, _N - _PW, w * _PSTRIDE)
  pltpu.sync_copy(batch_hbm.at[pl.ds(base, _PW)], bat_v)

  for l, h_hbm in enumerate((h0_hbm, h1_hbm, h2_hbm)):
    pltpu.sync_copy(h_hbm.at[pl.ds(base, _PW)], rows_v)

    def ibody(i, carry):
      for j in range(_D // 16):
        acc_v[i, pl.ds(j * 16, 16)] = jnp.full((16,), -jnp.inf, jnp.float32)
      return carry

    lax.fori_loop(0, _G, ibody, 0)

    def rbody(q, carry):
      gvec = bat_v[pl.ds(q * 16, 16)]
      for r in range(16):
        g = gvec[r]
        for j in range(_D // 16):
          sl = pl.ds(j * 16, 16)
          acc_v[g, sl] = jnp.maximum(acc_v[g, sl], rows_v[q * 16 + r, sl])
      return carry

    lax.fori_loop(0, _PW // 16, rbody, 0)
    pltpu.sync_copy(acc_v, out_hbm.at[l, w])


def _pool(h0, h1, h2, batch):
  mesh = plsc.VectorSubcoreMesh(core_axis_name="c", subcore_axis_name="s")
  return pl.kernel(
      _pool_body,
      out_type=jax.ShapeDtypeStruct((3, _NW, _G, _D), jnp.float32),
      mesh=mesh,
      scratch_types=[
          pltpu.VMEM((_PW,), jnp.int32),
          pltpu.VMEM((_PW, _D), jnp.float32),
          pltpu.VMEM((_G, _D), jnp.float32),
      ],
  )(h0, h1, h2, batch)


def _layer_body(h_ref, p_ref, w1_ref, b1_ref, w2_ref, b2_ref, g_ref, bb_ref,
                a_ref, out_ref):
  z = h_ref[...] + p_ref[0] + p_ref[1]
  t = jnp.dot(z, w1_ref[...], preferred_element_type=jnp.float32) + b1_ref[...]
  t = jnp.maximum(t, 0.0)
  u = jnp.dot(t, w2_ref[...], preferred_element_type=jnp.float32) + b2_ref[...]
  mu = jnp.mean(u, axis=0, keepdims=True)
  var = jnp.mean((u - mu) * (u - mu), axis=0, keepdims=True)
  v = g_ref[...] * (u - mu) * lax.rsqrt(var + 1e-5) + bb_ref[...]
  out_ref[...] = jnp.where(v >= 0.0, v, a_ref[...] * v)


def _layer(h, p, w1, b1, w2, b2, g, bb, a):
  return pl.pallas_call(
      _layer_body,
      out_shape=jax.ShapeDtypeStruct((_N, _D), jnp.float32),
  )(h, p, w1, b1.reshape(1, _D), w2, b2.reshape(1, _D), g.reshape(1, _D),
    bb.reshape(1, _D), jnp.full((1, _D), a, jnp.float32))


def _poollin_body(part_ref, w_ref, b_ref, out_ref):
  for l in range(3):
    acc = part_ref[l, 0]
    for k in range(1, _NW):
      acc = jnp.maximum(acc, part_ref[l, k])
    out_ref[l] = (
        jnp.dot(acc, w_ref[l], preferred_element_type=jnp.float32) + b_ref[l])


def _poollin(part, ws, bs):
  return pl.pallas_call(
      _poollin_body,
      out_shape=jax.ShapeDtypeStruct((3, _G, _D), jnp.float32),
  )(part, ws, bs)


def kernel(x, edge_index, batch, params):
  src = edge_index[0]
  dst = edge_index[1]

  h = x
  hidden = [x]
  for l in range(2):
    p = _seg_sum(h, src, dst)
    h = _layer(h, p, params[f"gin{l}_W1"], params[f"gin{l}_b1"],
               params[f"gin{l}_W2"], params[f"gin{l}_b2"],
               params[f"bn{l}_g"], params[f"bn{l}_b"], params["prelu_a"])
    hidden.append(h)

  part = _pool(hidden[0], hidden[1], hidden[2], batch)
  ws = jnp.stack([params[f"lin{l}_W"] for l in range(3)])
  bs = jnp.stack([params[f"lin{l}_b"] for l in range(3)]).reshape(3, 1, _D)
  reps = _poollin(part, ws, bs)
  return jnp.transpose(reps, (1, 2, 0)).reshape(_G, 3 * _D)


# speedup vs baseline: 9.8500x; 2.1343x over previous
import functools
import jax
import jax.numpy as jnp
from jax import lax
from jax.experimental import pallas as pl
from jax.experimental.pallas import tpu as pltpu
from jax.experimental.pallas import tpu_sc as plsc

_N = 10000
_E = 320000
_D = 128
_G = 64
_NC = 2
_NS = 16
_NW = _NC * _NS


_EPW = _E // _NW
_CHK = 80
_NCHK = _EPW // _CHK
_ZR = 40
_NZC = _N // _ZR


def _seg_sum_body(h_hbm, src_hbm, dst_hbm, out_hbm, src_v, dst0, dst1, rows0,
                  rows1, znc_v, acc_sh, isem0, isem1, gsem0, gsem1, ssem0,
                  ssem1):
  c = lax.axis_index("c")
  s = lax.axis_index("s")
  w = c * _NS + s
  rows = (rows0, rows1)
  dst = (dst0, dst1)
  isem = (isem0, isem1)
  gsem = (gsem0, gsem1)
  ssem = (ssem0, ssem1)
  ebase = w * _EPW

  pltpu.sync_copy(src_hbm.at[pl.ds(ebase, _EPW)], src_v)
  for b in range(2):
    pltpu.async_copy(dst_hbm.at[pl.ds(ebase + b * _CHK, _CHK)], dst[b],
                     isem[b])
    pltpu.async_copy(h_hbm.at[src_v.at[pl.ds(b * _CHK, _CHK)]], rows[b],
                     gsem[b])

  nck = jnp.where(s < _NZC - _NS * (_NZC // _NS), _NZC // _NS + 1, _NZC // _NS)

  def zbody(i, carry):
    for j in range(_D // 16):
      znc_v[i, pl.ds(j * 16, 16)] = jnp.zeros((16,), jnp.float32)
    return carry

  lax.fori_loop(0, _ZR, zbody, 0)

  def zcopy(k, carry):
    pltpu.sync_copy(znc_v, acc_sh.at[pl.ds((s + _NS * k) * _ZR, _ZR)])
    return carry

  lax.fori_loop(0, nck, zcopy, 0)
  plsc.subcore_barrier()

  def half(t, b):
    pltpu.make_async_copy(h_hbm.at[src_v.at[pl.ds(0, _CHK)]], rows[b],
                          gsem[b]).wait()
    pltpu.make_async_copy(dst_hbm.at[pl.ds(ebase, _CHK)], dst[b],
                          isem[b]).wait()
    pltpu.async_copy(rows[b], acc_sh.at[dst[b]], ssem[b], add=True)

    @pl.when(t + 2 < _NCHK)
    def _():
      pltpu.make_async_copy(rows[b], acc_sh.at[dst[b]], ssem[b]).wait()
      pltpu.async_copy(dst_hbm.at[pl.ds(ebase + (t + 2) * _CHK, _CHK)],
                       dst[b], isem[b])
      pltpu.async_copy(h_hbm.at[src_v.at[pl.ds((t + 2) * _CHK, _CHK)]],
                       rows[b], gsem[b])

  def body(j, carry):
    for b in range(2):
      half(2 * j + b, b)
    return carry

  lax.fori_loop(0, _NCHK // 2, body, 0)
  half(_NCHK - 1, 0)
  pltpu.make_async_copy(rows0, acc_sh.at[dst0], ssem0).wait()
  pltpu.make_async_copy(rows1, acc_sh.at[dst1], ssem1).wait()
  plsc.subcore_barrier()

  def dcopy(k, carry):
    r0 = (s + _NS * k) * _ZR
    pltpu.sync_copy(acc_sh.at[pl.ds(r0, _ZR)], znc_v)
    pltpu.sync_copy(znc_v, out_hbm.at[c, pl.ds(r0, _ZR)])
    return carry

  lax.fori_loop(0, nck, dcopy, 0)


def _seg_sum(h, src, dst):
  mesh = plsc.VectorSubcoreMesh(core_axis_name="c", subcore_axis_name="s")
  return pl.kernel(
      _seg_sum_body,
      out_type=jax.ShapeDtypeStruct((_NC, _N, _D), jnp.float32),
      mesh=mesh,
      scratch_types=[
          pltpu.VMEM((_EPW,), jnp.int32),
          pltpu.VMEM((_CHK,), jnp.int32),
          pltpu.VMEM((_CHK,), jnp.int32),
          pltpu.VMEM((_CHK, _D), jnp.float32),
          pltpu.VMEM((_CHK, _D), jnp.float32),
          pltpu.VMEM((_ZR, _D), jnp.float32),
          pltpu.VMEM_SHARED((_N, _D), jnp.float32),
          pltpu.SemaphoreType.DMA,
          pltpu.SemaphoreType.DMA,
          pltpu.SemaphoreType.DMA,
          pltpu.SemaphoreType.DMA,
          pltpu.SemaphoreType.DMA,
          pltpu.SemaphoreType.DMA,
      ],
  )(h, src, dst)


_PW = 320
_PSTRIDE = 312


def _pool_body(h0_hbm, h1_hbm, h2_hbm, batch_hbm, out_hbm, bat_v, rows_v,
               acc_v):
  c = lax.axis_index("c")
  s = lax.axis_index("s")
  w = c * _NS + s
  base = jnp.where(w == _NW - 1, _N - _PW, w * _PSTRIDE)
  pltpu.sync_copy(batch_hbm.at[pl.ds(base, _PW)], bat_v)

  for l, h_hbm in enumerate((h0_hbm, h1_hbm, h2_hbm)):
    pltpu.sync_copy(h_hbm.at[pl.ds(base, _PW)], rows_v)

    def ibody(i, carry):
      for j in range(_D // 16):
        acc_v[i, pl.ds(j * 16, 16)] = jnp.full((16,), -jnp.inf, jnp.float32)
      return carry

    lax.fori_loop(0, _G, ibody, 0)

    def rbody(q, carry):
      gvec = bat_v[pl.ds(q * 16, 16)]
      for r in range(16):
        g = gvec[r]
        for j in range(_D // 16):
          sl = pl.ds(j * 16, 16)
          acc_v[g, sl] = jnp.maximum(acc_v[g, sl], rows_v[q * 16 + r, sl])
      return carry

    lax.fori_loop(0, _PW // 16, rbody, 0)
    pltpu.sync_copy(acc_v, out_hbm.at[l, w])


def _pool(h0, h1, h2, batch):
  mesh = plsc.VectorSubcoreMesh(core_axis_name="c", subcore_axis_name="s")
  return pl.kernel(
      _pool_body,
      out_type=jax.ShapeDtypeStruct((3, _NW, _G, _D), jnp.float32),
      mesh=mesh,
      scratch_types=[
          pltpu.VMEM((_PW,), jnp.int32),
          pltpu.VMEM((_PW, _D), jnp.float32),
          pltpu.VMEM((_G, _D), jnp.float32),
      ],
  )(h0, h1, h2, batch)


def _layer_body(h_ref, p_ref, w1_ref, b1_ref, w2_ref, b2_ref, g_ref, bb_ref,
                a_ref, out_ref):
  z = h_ref[...] + p_ref[0] + p_ref[1]
  t = jnp.dot(z, w1_ref[...], preferred_element_type=jnp.float32) + b1_ref[...]
  t = jnp.maximum(t, 0.0)
  u = jnp.dot(t, w2_ref[...], preferred_element_type=jnp.float32) + b2_ref[...]
  mu = jnp.mean(u, axis=0, keepdims=True)
  var = jnp.mean((u - mu) * (u - mu), axis=0, keepdims=True)
  v = g_ref[...] * (u - mu) * lax.rsqrt(var + 1e-5) + bb_ref[...]
  out_ref[...] = jnp.where(v >= 0.0, v, a_ref[...] * v)


def _layer(h, p, w1, b1, w2, b2, g, bb, a):
  return pl.pallas_call(
      _layer_body,
      out_shape=jax.ShapeDtypeStruct((_N, _D), jnp.float32),
  )(h, p, w1, b1.reshape(1, _D), w2, b2.reshape(1, _D), g.reshape(1, _D),
    bb.reshape(1, _D), jnp.full((1, _D), a, jnp.float32))


def _poollin_body(part_ref, w_ref, b_ref, out_ref):
  for l in range(3):
    acc = part_ref[l, 0]
    for k in range(1, _NW):
      acc = jnp.maximum(acc, part_ref[l, k])
    out_ref[l] = (
        jnp.dot(acc, w_ref[l], preferred_element_type=jnp.float32) + b_ref[l])


def _poollin(part, ws, bs):
  return pl.pallas_call(
      _poollin_body,
      out_shape=jax.ShapeDtypeStruct((3, _G, _D), jnp.float32),
  )(part, ws, bs)


def kernel(x, edge_index, batch, params):
  src = edge_index[0]
  dst = edge_index[1]

  h = x
  hidden = [x]
  for l in range(2):
    p = _seg_sum(h, src, dst)
    h = _layer(h, p, params[f"gin{l}_W1"], params[f"gin{l}_b1"],
               params[f"gin{l}_W2"], params[f"gin{l}_b2"],
               params[f"bn{l}_g"], params[f"bn{l}_b"], params["prelu_a"])
    hidden.append(h)

  part = _pool(hidden[0], hidden[1], hidden[2], batch)
  ws = jnp.stack([params[f"lin{l}_W"] for l in range(3)])
  bs = jnp.stack([params[f"lin{l}_b"] for l in range(3)]).reshape(3, 1, _D)
  reps = _poollin(part, ws, bs)
  return jnp.transpose(reps, (1, 2, 0)).reshape(_G, 3 * _D)


# speedup vs baseline: 10.0050x; 1.0157x over previous
import functools
import jax
import jax.numpy as jnp
from jax import lax
from jax.experimental import pallas as pl
from jax.experimental.pallas import tpu as pltpu
from jax.experimental.pallas import tpu_sc as plsc

_N = 10000
_E = 320000
_D = 128
_G = 64
_NC = 2
_NS = 16
_NW = _NC * _NS


_EPW = _E // _NW
_CHK = 80
_NCHK = _EPW // _CHK
_ZR = 40
_NZC = _N // _ZR


def _seg_sum_body(h_hbm, src_hbm, dst_hbm, out_hbm, src_v, dst0, dst1, rows0,
                  rows1, znc_v, acc_sh, isem0, isem1, gsem0, gsem1, ssem0,
                  ssem1):
  c = lax.axis_index("c")
  s = lax.axis_index("s")
  w = c * _NS + s
  rows = (rows0, rows1)
  dst = (dst0, dst1)
  isem = (isem0, isem1)
  gsem = (gsem0, gsem1)
  ssem = (ssem0, ssem1)
  ebase = w * _EPW

  pltpu.sync_copy(src_hbm.at[pl.ds(ebase, _EPW)], src_v)
  for b in range(2):
    pltpu.async_copy(dst_hbm.at[pl.ds(ebase + b * _CHK, _CHK)], dst[b],
                     isem[b])
    pltpu.async_copy(h_hbm.at[src_v.at[pl.ds(b * _CHK, _CHK)]], rows[b],
                     gsem[b])

  nck = jnp.where(s < _NZC - _NS * (_NZC // _NS), _NZC // _NS + 1, _NZC // _NS)

  def zbody(i, carry):
    for j in range(_D // 16):
      znc_v[i, pl.ds(j * 16, 16)] = jnp.zeros((16,), jnp.float32)
    return carry

  lax.fori_loop(0, _ZR, zbody, 0)

  def zcopy(k, carry):
    pltpu.async_copy(znc_v, acc_sh.at[pl.ds((s + _NS * k) * _ZR, _ZR)], ssem0)
    return carry

  lax.fori_loop(0, nck, zcopy, 0)

  def zdrain(k, carry):
    pltpu.make_async_copy(znc_v, acc_sh.at[pl.ds(0, _ZR)], ssem0).wait()
    return carry

  lax.fori_loop(0, nck, zdrain, 0)
  plsc.subcore_barrier()

  def half(t, b):
    pltpu.make_async_copy(h_hbm.at[src_v.at[pl.ds(0, _CHK)]], rows[b],
                          gsem[b]).wait()
    pltpu.make_async_copy(dst_hbm.at[pl.ds(ebase, _CHK)], dst[b],
                          isem[b]).wait()
    pltpu.async_copy(rows[b], acc_sh.at[dst[b]], ssem[b], add=True)

    @pl.when(t + 2 < _NCHK)
    def _():
      pltpu.make_async_copy(rows[b], acc_sh.at[dst[b]], ssem[b]).wait()
      pltpu.async_copy(dst_hbm.at[pl.ds(ebase + (t + 2) * _CHK, _CHK)],
                       dst[b], isem[b])
      pltpu.async_copy(h_hbm.at[src_v.at[pl.ds((t + 2) * _CHK, _CHK)]],
                       rows[b], gsem[b])

  def body(j, carry):
    for b in range(2):
      half(2 * j + b, b)
    return carry

  lax.fori_loop(0, _NCHK // 2, body, 0)
  half(_NCHK - 1, 0)
  pltpu.make_async_copy(rows0, acc_sh.at[dst0], ssem0).wait()
  pltpu.make_async_copy(rows1, acc_sh.at[dst1], ssem1).wait()
  plsc.subcore_barrier()

  def dcopy(k, carry):
    r0 = (s + _NS * k) * _ZR
    pltpu.async_copy(acc_sh.at[pl.ds(r0, _ZR)], out_hbm.at[c, pl.ds(r0, _ZR)],
                     isem0)
    return carry

  lax.fori_loop(0, nck, dcopy, 0)

  def ddrain(k, carry):
    pltpu.make_async_copy(acc_sh.at[pl.ds(0, _ZR)],
                          out_hbm.at[c, pl.ds(0, _ZR)], isem0).wait()
    return carry

  lax.fori_loop(0, nck, ddrain, 0)


def _seg_sum(h, src, dst):
  mesh = plsc.VectorSubcoreMesh(core_axis_name="c", subcore_axis_name="s")
  return pl.kernel(
      _seg_sum_body,
      out_type=jax.ShapeDtypeStruct((_NC, _N, _D), jnp.float32),
      mesh=mesh,
      scratch_types=[
          pltpu.VMEM((_EPW,), jnp.int32),
          pltpu.VMEM((_CHK,), jnp.int32),
          pltpu.VMEM((_CHK,), jnp.int32),
          pltpu.VMEM((_CHK, _D), jnp.float32),
          pltpu.VMEM((_CHK, _D), jnp.float32),
          pltpu.VMEM((_ZR, _D), jnp.float32),
          pltpu.VMEM_SHARED((_N, _D), jnp.float32),
          pltpu.SemaphoreType.DMA,
          pltpu.SemaphoreType.DMA,
          pltpu.SemaphoreType.DMA,
          pltpu.SemaphoreType.DMA,
          pltpu.SemaphoreType.DMA,
          pltpu.SemaphoreType.DMA,
      ],
  )(h, src, dst)


_PW = 320
_PSTRIDE = 312


def _pool_body(h0_hbm, h1_hbm, h2_hbm, batch_hbm, out_hbm, bat_v, rows_v,
               acc_v):
  c = lax.axis_index("c")
  s = lax.axis_index("s")
  w = c * _NS + s
  base = jnp.where(w == _NW - 1, _N - _PW, w * _PSTRIDE)
  pltpu.sync_copy(batch_hbm.at[pl.ds(base, _PW)], bat_v)

  for l, h_hbm in enumerate((h0_hbm, h1_hbm, h2_hbm)):
    pltpu.sync_copy(h_hbm.at[pl.ds(base, _PW)], rows_v)

    def ibody(i, carry):
      for j in range(_D // 16):
        acc_v[i, pl.ds(j * 16, 16)] = jnp.full((16,), -jnp.inf, jnp.float32)
      return carry

    lax.fori_loop(0, _G, ibody, 0)

    def rbody(q, carry):
      gvec = bat_v[pl.ds(q * 16, 16)]
      for r in range(16):
        g = gvec[r]
        for j in range(_D // 16):
          sl = pl.ds(j * 16, 16)
          acc_v[g, sl] = jnp.maximum(acc_v[g, sl], rows_v[q * 16 + r, sl])
      return carry

    lax.fori_loop(0, _PW // 16, rbody, 0)
    pltpu.sync_copy(acc_v, out_hbm.at[l, w])


def _pool(h0, h1, h2, batch):
  mesh = plsc.VectorSubcoreMesh(core_axis_name="c", subcore_axis_name="s")
  return pl.kernel(
      _pool_body,
      out_type=jax.ShapeDtypeStruct((3, _NW, _G, _D), jnp.float32),
      mesh=mesh,
      scratch_types=[
          pltpu.VMEM((_PW,), jnp.int32),
          pltpu.VMEM((_PW, _D), jnp.float32),
          pltpu.VMEM((_G, _D), jnp.float32),
      ],
  )(h0, h1, h2, batch)


def _layer_body(h_ref, p_ref, w1_ref, b1_ref, w2_ref, b2_ref, g_ref, bb_ref,
                a_ref, out_ref):
  z = h_ref[...] + p_ref[0] + p_ref[1]
  t = jnp.dot(z, w1_ref[...], preferred_element_type=jnp.float32) + b1_ref[...]
  t = jnp.maximum(t, 0.0)
  u = jnp.dot(t, w2_ref[...], preferred_element_type=jnp.float32) + b2_ref[...]
  mu = jnp.mean(u, axis=0, keepdims=True)
  var = jnp.mean((u - mu) * (u - mu), axis=0, keepdims=True)
  v = g_ref[...] * (u - mu) * lax.rsqrt(var + 1e-5) + bb_ref[...]
  out_ref[...] = jnp.where(v >= 0.0, v, a_ref[...] * v)


def _layer(h, p, w1, b1, w2, b2, g, bb, a):
  return pl.pallas_call(
      _layer_body,
      out_shape=jax.ShapeDtypeStruct((_N, _D), jnp.float32),
  )(h, p, w1, b1.reshape(1, _D), w2, b2.reshape(1, _D), g.reshape(1, _D),
    bb.reshape(1, _D), jnp.full((1, _D), a, jnp.float32))


def _poollin_body(part_ref, w_ref, b_ref, out_ref):
  for l in range(3):
    acc = part_ref[l, 0]
    for k in range(1, _NW):
      acc = jnp.maximum(acc, part_ref[l, k])
    out_ref[l] = (
        jnp.dot(acc, w_ref[l], preferred_element_type=jnp.float32) + b_ref[l])


def _poollin(part, ws, bs):
  return pl.pallas_call(
      _poollin_body,
      out_shape=jax.ShapeDtypeStruct((3, _G, _D), jnp.float32),
  )(part, ws, bs)


def kernel(x, edge_index, batch, params):
  src = edge_index[0]
  dst = edge_index[1]

  h = x
  hidden = [x]
  for l in range(2):
    p = _seg_sum(h, src, dst)
    h = _layer(h, p, params[f"gin{l}_W1"], params[f"gin{l}_b1"],
               params[f"gin{l}_W2"], params[f"gin{l}_b2"],
               params[f"bn{l}_g"], params[f"bn{l}_b"], params["prelu_a"])
    hidden.append(h)

  part = _pool(hidden[0], hidden[1], hidden[2], batch)
  ws = jnp.stack([params[f"lin{l}_W"] for l in range(3)])
  bs = jnp.stack([params[f"lin{l}_b"] for l in range(3)]).reshape(3, 1, _D)
  reps = _poollin(part, ws, bs)
  return jnp.transpose(reps, (1, 2, 0)).reshape(_G, 3 * _D)


# speedup vs baseline: 11.5357x; 1.1530x over previous
import functools
import jax
import jax.numpy as jnp
from jax import lax
from jax.experimental import pallas as pl
from jax.experimental.pallas import tpu as pltpu
from jax.experimental.pallas import tpu_sc as plsc

_N = 10000
_E = 320000
_D = 128
_G = 64
_NC = 2
_NS = 16
_NW = _NC * _NS


_EPW = _E // _NW
_CHK = 40
_NCHK = _EPW // _CHK
_NBUF = 4
_ZR = 40
_NZC = _N // _ZR


def _seg_sum_body(h_hbm, src_hbm, dst_hbm, out_hbm, src_v, dsts, rowss,
                  znc_v, acc_sh, isems, gsems, ssems):
  c = lax.axis_index("c")
  s = lax.axis_index("s")
  w = c * _NS + s
  rows = tuple(rowss)
  dst = tuple(dsts)
  isem = tuple(isems)
  gsem = tuple(gsems)
  ssem = tuple(ssems)
  ebase = w * _EPW

  pltpu.sync_copy(src_hbm.at[pl.ds(ebase, _EPW)], src_v)
  for b in range(_NBUF):
    pltpu.async_copy(dst_hbm.at[pl.ds(ebase + b * _CHK, _CHK)], dst[b],
                     isem[b])
    pltpu.async_copy(h_hbm.at[src_v.at[pl.ds(b * _CHK, _CHK)]], rows[b],
                     gsem[b])

  nck = jnp.where(s < _NZC - _NS * (_NZC // _NS), _NZC // _NS + 1, _NZC // _NS)

  def zbody(i, carry):
    for j in range(_D // 16):
      znc_v[i, pl.ds(j * 16, 16)] = jnp.zeros((16,), jnp.float32)
    return carry

  lax.fori_loop(0, _ZR, zbody, 0)

  def zcopy(k, carry):
    pltpu.async_copy(znc_v, acc_sh.at[pl.ds((s + _NS * k) * _ZR, _ZR)],
                     ssem[0])
    return carry

  lax.fori_loop(0, nck, zcopy, 0)

  def zdrain(k, carry):
    pltpu.make_async_copy(znc_v, acc_sh.at[pl.ds(0, _ZR)], ssem[0]).wait()
    return carry

  lax.fori_loop(0, nck, zdrain, 0)
  plsc.subcore_barrier()

  def stage(t, b):
    pltpu.make_async_copy(h_hbm.at[src_v.at[pl.ds(0, _CHK)]], rows[b],
                          gsem[b]).wait()
    pltpu.make_async_copy(dst_hbm.at[pl.ds(ebase, _CHK)], dst[b],
                          isem[b]).wait()
    pltpu.async_copy(rows[b], acc_sh.at[dst[b]], ssem[b], add=True)

    @pl.when(t + _NBUF < _NCHK)
    def _():
      pltpu.make_async_copy(rows[b], acc_sh.at[dst[b]], ssem[b]).wait()
      pltpu.async_copy(dst_hbm.at[pl.ds(ebase + (t + _NBUF) * _CHK, _CHK)],
                       dst[b], isem[b])
      pltpu.async_copy(h_hbm.at[src_v.at[pl.ds((t + _NBUF) * _CHK, _CHK)]],
                       rows[b], gsem[b])

  def body(j, carry):
    for b in range(_NBUF):
      stage(_NBUF * j + b, b)
    return carry

  _NFULL = _NCHK // _NBUF
  lax.fori_loop(0, _NFULL, body, 0)
  for t in range(_NFULL * _NBUF, _NCHK):
    stage(t, t % _NBUF)
  for b in range(_NBUF):
    pltpu.make_async_copy(rows[b], acc_sh.at[dst[b]], ssem[b]).wait()
  plsc.subcore_barrier()

  def dcopy(k, carry):
    r0 = (s + _NS * k) * _ZR
    pltpu.async_copy(acc_sh.at[pl.ds(r0, _ZR)], out_hbm.at[c, pl.ds(r0, _ZR)],
                     isem[0])
    return carry

  lax.fori_loop(0, nck, dcopy, 0)

  def ddrain(k, carry):
    pltpu.make_async_copy(acc_sh.at[pl.ds(0, _ZR)],
                          out_hbm.at[c, pl.ds(0, _ZR)], isem[0]).wait()
    return carry

  lax.fori_loop(0, nck, ddrain, 0)


def _seg_sum(h, src, dst):
  mesh = plsc.VectorSubcoreMesh(core_axis_name="c", subcore_axis_name="s")
  return pl.kernel(
      _seg_sum_body,
      out_type=jax.ShapeDtypeStruct((_NC, _N, _D), jnp.float32),
      mesh=mesh,
      scratch_types=[
          pltpu.VMEM((_EPW,), jnp.int32),
          [pltpu.VMEM((_CHK,), jnp.int32) for _ in range(_NBUF)],
          [pltpu.VMEM((_CHK, _D), jnp.float32) for _ in range(_NBUF)],
          pltpu.VMEM((_ZR, _D), jnp.float32),
          pltpu.VMEM_SHARED((_N, _D), jnp.float32),
          [pltpu.SemaphoreType.DMA for _ in range(_NBUF)],
          [pltpu.SemaphoreType.DMA for _ in range(_NBUF)],
          [pltpu.SemaphoreType.DMA for _ in range(_NBUF)],
      ],
  )(h, src, dst)


_PW = 320
_PSTRIDE = 312


def _pool_body(h0_hbm, h1_hbm, h2_hbm, batch_hbm, out_hbm, bat_v, rows_v,
               acc_v):
  c = lax.axis_index("c")
  s = lax.axis_index("s")
  w = c * _NS + s
  base = jnp.where(w == _NW - 1, _N - _PW, w * _PSTRIDE)
  pltpu.sync_copy(batch_hbm.at[pl.ds(base, _PW)], bat_v)

  for l, h_hbm in enumerate((h0_hbm, h1_hbm, h2_hbm)):
    pltpu.sync_copy(h_hbm.at[pl.ds(base, _PW)], rows_v)

    def ibody(i, carry):
      for j in range(_D // 16):
        acc_v[i, pl.ds(j * 16, 16)] = jnp.full((16,), -jnp.inf, jnp.float32)
      return carry

    lax.fori_loop(0, _G, ibody, 0)

    def rbody(q, carry):
      gvec = bat_v[pl.ds(q * 16, 16)]
      for r in range(16):
        g = gvec[r]
        for j in range(_D // 16):
          sl = pl.ds(j * 16, 16)
          acc_v[g, sl] = jnp.maximum(acc_v[g, sl], rows_v[q * 16 + r, sl])
      return carry

    lax.fori_loop(0, _PW // 16, rbody, 0)
    pltpu.sync_copy(acc_v, out_hbm.at[l, w])


def _pool(h0, h1, h2, batch):
  mesh = plsc.VectorSubcoreMesh(core_axis_name="c", subcore_axis_name="s")
  return pl.kernel(
      _pool_body,
      out_type=jax.ShapeDtypeStruct((3, _NW, _G, _D), jnp.float32),
      mesh=mesh,
      scratch_types=[
          pltpu.VMEM((_PW,), jnp.int32),
          pltpu.VMEM((_PW, _D), jnp.float32),
          pltpu.VMEM((_G, _D), jnp.float32),
      ],
  )(h0, h1, h2, batch)


def _layer_body(h_ref, p_ref, w1_ref, b1_ref, w2_ref, b2_ref, g_ref, bb_ref,
                a_ref, out_ref):
  z = h_ref[...] + p_ref[0] + p_ref[1]
  t = jnp.dot(z, w1_ref[...], preferred_element_type=jnp.float32) + b1_ref[...]
  t = jnp.maximum(t, 0.0)
  u = jnp.dot(t, w2_ref[...], preferred_element_type=jnp.float32) + b2_ref[...]
  mu = jnp.mean(u, axis=0, keepdims=True)
  var = jnp.mean((u - mu) * (u - mu), axis=0, keepdims=True)
  v = g_ref[...] * (u - mu) * lax.rsqrt(var + 1e-5) + bb_ref[...]
  out_ref[...] = jnp.where(v >= 0.0, v, a_ref[...] * v)


def _layer(h, p, w1, b1, w2, b2, g, bb, a):
  return pl.pallas_call(
      _layer_body,
      out_shape=jax.ShapeDtypeStruct((_N, _D), jnp.float32),
  )(h, p, w1, b1.reshape(1, _D), w2, b2.reshape(1, _D), g.reshape(1, _D),
    bb.reshape(1, _D), jnp.full((1, _D), a, jnp.float32))


def _poollin_body(part_ref, w_ref, b_ref, out_ref):
  for l in range(3):
    acc = part_ref[l, 0]
    for k in range(1, _NW):
      acc = jnp.maximum(acc, part_ref[l, k])
    out_ref[l] = (
        jnp.dot(acc, w_ref[l], preferred_element_type=jnp.float32) + b_ref[l])


def _poollin(part, ws, bs):
  return pl.pallas_call(
      _poollin_body,
      out_shape=jax.ShapeDtypeStruct((3, _G, _D), jnp.float32),
  )(part, ws, bs)


def kernel(x, edge_index, batch, params):
  src = edge_index[0]
  dst = edge_index[1]

  h = x
  hidden = [x]
  for l in range(2):
    p = _seg_sum(h, src, dst)
    h = _layer(h, p, params[f"gin{l}_W1"], params[f"gin{l}_b1"],
               params[f"gin{l}_W2"], params[f"gin{l}_b2"],
               params[f"bn{l}_g"], params[f"bn{l}_b"], params["prelu_a"])
    hidden.append(h)

  part = _pool(hidden[0], hidden[1], hidden[2], batch)
  ws = jnp.stack([params[f"lin{l}_W"] for l in range(3)])
  bs = jnp.stack([params[f"lin{l}_b"] for l in range(3)]).reshape(3, 1, _D)
  reps = _poollin(part, ws, bs)
  return jnp.transpose(reps, (1, 2, 0)).reshape(_G, 3 * _D)
